# baseline (speedup 1.0000x reference)
import functools

import jax
import jax.numpy as jnp
from jax import lax
from jax.experimental import pallas as pl
from jax.experimental.pallas import tpu as pltpu
from jax.experimental.pallas import tpu_sc as plsc

PAD_ID = 999999
D = 64

NC = 2
NS = 16
NW = NC * NS

NBUF = 4


def _gather_body(n_chunks, chunk, idx_hbm, table_hbm, out_hbm,
                 idx_v, rows, gsems, osems):
    wid = lax.axis_index("s") * NC + lax.axis_index("c")
    bpw = n_chunks * chunk
    base = wid * bpw

    pltpu.sync_copy(idx_hbm.at[wid], idx_v)

    def gather(g, j):
        return pltpu.async_copy(table_hbm.at[idx_v.at[g]], rows[j], gsems[j])

    def out_slice(g):
        return out_hbm.at[pl.ds(base + g * chunk, chunk)]

    def put(g, j):
        return pltpu.async_copy(rows[j], out_slice(g), osems[j])

    def wait_gather(g, j):
        pltpu.make_async_copy(table_hbm.at[idx_v.at[g]], rows[j],
                              gsems[j]).wait()

    def wait_put(g, j):
        pltpu.make_async_copy(rows[j], out_slice(g), osems[j]).wait()

    for j in range(NBUF):
        gather(j, j)

    @pl.loop(0, n_chunks - NBUF, step=NBUF)
    def _(g0):
        for j in range(NBUF):
            g = g0 + j
            wait_gather(g, j)
            put(g, j)
            wait_put(g, j)
            gather(g + NBUF, j)

    for j in range(NBUF):
        g = n_chunks - NBUF + j
        wait_gather(g, j)
        put(g, j)
    for j in range(NBUF):
        wait_put(n_chunks - NBUF + j, j)


def _mask_body(idx_ref, out_ref):
    out_ref[...] = idx_ref[...] == PAD_ID


@jax.jit
def kernel(word_indices, vocabulary):
    n_rows, seq = word_indices.shape
    b = n_rows * seq
    bpw = b // NW
    chunk = 128
    n_chunks = bpw // chunk

    idx_flat = word_indices.reshape(NW, n_chunks, chunk)

    mesh = plsc.VectorSubcoreMesh(core_axis_name="c", subcore_axis_name="s")
    gathered = pl.kernel(
        functools.partial(_gather_body, n_chunks, chunk),
        out_type=jax.ShapeDtypeStruct((b, D), jnp.float32),
        mesh=mesh,
        scratch_types=[
            pltpu.VMEM((n_chunks, chunk), jnp.int32),
            tuple(pltpu.VMEM((chunk, D), jnp.float32) for _ in range(NBUF)),
            tuple(pltpu.SemaphoreType.DMA for _ in range(NBUF)),
            tuple(pltpu.SemaphoreType.DMA for _ in range(NBUF)),
        ],
        compiler_params=pltpu.CompilerParams(use_tc_tiling_on_sc=False),
    )(idx_flat, vocabulary)

    mask = pl.pallas_call(
        _mask_body,
        out_shape=jax.ShapeDtypeStruct((n_rows, seq), jnp.bool_),
    )(word_indices)

    return gathered.reshape(n_rows, seq, D), mask


# speedup vs baseline: 1.0006x; 1.0006x over previous
import functools

import jax
import jax.numpy as jnp
from jax import lax
from jax.experimental import pallas as pl
from jax.experimental.pallas import tpu as pltpu
from jax.experimental.pallas import tpu_sc as plsc

PAD_ID = 999999
D = 64

NC = 2
NS = 16
NW = NC * NS

NBUF = 8


def _gather_body(n_chunks, chunk, idx_hbm, table_hbm, out_hbm,
                 idx_v, rows, gsems, osems):
    wid = lax.axis_index("s") * NC + lax.axis_index("c")
    bpw = n_chunks * chunk
    base = wid * bpw

    pltpu.sync_copy(idx_hbm.at[wid], idx_v)

    def gather(g, j):
        return pltpu.async_copy(table_hbm.at[idx_v.at[g]], rows[j], gsems[j])

    def out_slice(g):
        return out_hbm.at[pl.ds(base + g * chunk, chunk)]

    def put(g, j):
        return pltpu.async_copy(rows[j], out_slice(g), osems[j])

    def wait_gather(g, j):
        pltpu.make_async_copy(table_hbm.at[idx_v.at[g]], rows[j],
                              gsems[j]).wait()

    def wait_put(g, j):
        pltpu.make_async_copy(rows[j], out_slice(g), osems[j]).wait()

    for j in range(NBUF):
        gather(j, j)

    @pl.loop(0, n_chunks - NBUF, step=NBUF)
    def _(g0):
        for j in range(NBUF):
            g = g0 + j
            wait_gather(g, j)
            put(g, j)
            wait_put(g, j)
            gather(g + NBUF, j)

    for j in range(NBUF):
        g = n_chunks - NBUF + j
        wait_gather(g, j)
        put(g, j)
    for j in range(NBUF):
        wait_put(n_chunks - NBUF + j, j)


def _mask_body(idx_ref, out_ref):
    out_ref[...] = idx_ref[...] == PAD_ID


@jax.jit
def kernel(word_indices, vocabulary):
    n_rows, seq = word_indices.shape
    b = n_rows * seq
    bpw = b // NW
    chunk = 128
    n_chunks = bpw // chunk

    idx_flat = word_indices.reshape(NW, n_chunks, chunk)

    mesh = plsc.VectorSubcoreMesh(core_axis_name="c", subcore_axis_name="s")
    gathered = pl.kernel(
        functools.partial(_gather_body, n_chunks, chunk),
        out_type=jax.ShapeDtypeStruct((b, D), jnp.float32),
        mesh=mesh,
        scratch_types=[
            pltpu.VMEM((n_chunks, chunk), jnp.int32),
            tuple(pltpu.VMEM((chunk, D), jnp.float32) for _ in range(NBUF)),
            tuple(pltpu.SemaphoreType.DMA for _ in range(NBUF)),
            tuple(pltpu.SemaphoreType.DMA for _ in range(NBUF)),
        ],
        compiler_params=pltpu.CompilerParams(use_tc_tiling_on_sc=False),
    )(idx_flat, vocabulary)

    mask = pl.pallas_call(
        _mask_body,
        out_shape=jax.ShapeDtypeStruct((n_rows, seq), jnp.bool_),
    )(word_indices)

    return gathered.reshape(n_rows, seq, D), mask
